# Initial kernel scaffold; baseline (speedup 1.0000x reference)
import jax
import jax.numpy as jnp
from jax.experimental import pallas as pl


def kernel(act, active_units):
    raise NotImplementedError("write your pallas kernel here")


# speedup vs baseline: 2.7947x; 2.7947x over previous
import jax
import jax.numpy as jnp
import numpy as np
from jax import lax
from jax.experimental import pallas as pl
from jax.experimental.pallas import tpu as pltpu
from jax.experimental.pallas import tpu_sc as plsc

N = 1_000_000
K = 10_000
NW = 16
C = 62_496
C15 = 62_560
NV = C // 16
NV15 = C15 // 16
CH = 744
CHV = CH // 16
CAP = 10_016
CAPS = 10_240

_I32MIN = np.int32(-(2**31))
_I32 = np.int32


def _zero_ref(ref, nv):
  z16 = lax.iota(jnp.int32, 16) * 0
  def z(i, _):
    ref[pl.ds(i * 16, 16)] = z16
    return 0
  lax.fori_loop(0, nv, z, 0)


def _hist_update(hist, bucket, mask):
  cnt, last = plsc.scan_count(bucket, mask)
  plsc.addupdate_scatter(hist, [bucket], cnt, mask=last)


def _body(act_hbm, active_hbm, out_hbm, win_hbm,
          keys, abuf, hist, row, ctlv, mbuf, cak, cai, cbk, cbi, smem,
          HG, CTL, TOF, MC, CSK, CSI):
  w = lax.axis_index("s")
  is15 = w == NW - 1
  base = pl.multiple_of(w * C, 8)
  hg_off = pl.multiple_of(w * 2048, 8)
  nv = jnp.where(is15, NV15, NV)

  @pl.when(jnp.logical_not(is15))
  def _():
    pltpu.sync_copy(act_hbm.at[pl.ds(base, C)], keys.at[pl.ds(0, C)])

  @pl.when(is15)
  def _():
    pltpu.sync_copy(act_hbm.at[pl.ds(base, C15)], keys)

  def keyify_vreg(o, a_act, a_active):
    a = a_act * a_active
    u = lax.bitcast_convert_type(a, jnp.int32)
    skey = jnp.where(u >= 0, u, u ^ _I32(0x7FFFFFFF))
    keys[pl.ds(o, 16)] = lax.bitcast_convert_type(skey, jnp.float32)

  def keyify_chunk(c, _):
    off = c * CH
    pltpu.sync_copy(active_hbm.at[pl.ds(base + off, CH)], abuf)
    def v(i, _):
      o = off + i * 16
      keyify_vreg(o, keys[pl.ds(o, 16)], abuf[pl.ds(i * 16, 16)])
      return 0
    lax.fori_loop(0, CHV, v, 0)
    return 0
  lax.fori_loop(0, C // CH, keyify_chunk, 0)

  @pl.when(is15)
  def _():
    pltpu.sync_copy(active_hbm.at[pl.ds(base + C, 64)], abuf.at[pl.ds(0, 64)])
    def v(i, _):
      o = C + i * 16
      keyify_vreg(o, keys[pl.ds(o, 16)], abuf[pl.ds(i * 16, 16)])
      return 0
    lax.fori_loop(0, 4, v, 0)

  def load_skey(i):
    return lax.bitcast_convert_type(keys[pl.ds(i * 16, 16)], jnp.int32)

  def merge_and_find(bins, cnt_gt):
    _zero_ref(hist, bins // 16)
    def addrow(ww, _):
      pltpu.sync_copy(HG.at[pl.ds(pl.multiple_of(ww * 2048, 8), bins)],
                      row.at[pl.ds(0, bins)])
      def av(i, _):
        hist[pl.ds(i * 16, 16)] = hist[pl.ds(i * 16, 16)] + row[pl.ds(i * 16, 16)]
        return 0
      lax.fori_loop(0, bins // 16, av, 0)
      return 0
    lax.fori_loop(0, NW, addrow, 0)

    def scan(j, carry):
      fb, fa, above = carry
      vb = bins // 16 - 1 - j
      v = hist[pl.ds(vb * 16, 16)]
      cs = plsc.cumsum(v)
      tot = jnp.max(cs)
      suf = above + tot - cs
      cond = jnp.logical_and(cnt_gt + suf < K, K <= cnt_gt + suf + v)
      lanei = lax.iota(jnp.int32, 16) + vb * 16
      fb = jnp.maximum(fb, jnp.max(jnp.where(cond, lanei, -1)))
      fa = jnp.maximum(fa, jnp.max(jnp.where(cond, suf, -1)))
      return fb, fa, above + tot
    B, above, _ = lax.fori_loop(0, bins // 16, scan,
                                (_I32(-1), _I32(-1), _I32(0)))
    return B, cnt_gt + above

  def publish_ctl(r, val):
    ctlv[pl.ds(r * 16, 16)] = jnp.full((16,), val, jnp.int32)
    pltpu.sync_copy(ctlv.at[pl.ds(r * 16, 16)], CTL.at[pl.ds(r * 16, 16)])

  _zero_ref(hist, 128)
  def h1(i, _):
    uk = load_skey(i) ^ _I32MIN
    _hist_update(hist, lax.shift_right_logical(uk, 21), None)
    return 0
  lax.fori_loop(0, nv, h1, 0)
  pltpu.sync_copy(hist.at[pl.ds(0, 2048)], HG.at[pl.ds(hg_off, 2048)])
  plsc.subcore_barrier()

  @pl.when(w == 0)
  def _():
    B1, cnt_gt = merge_and_find(2048, _I32(0))
    smem[0] = cnt_gt
    smem[1] = B1
    publish_ctl(0, B1)
  plsc.subcore_barrier()

  pltpu.sync_copy(CTL.at[pl.ds(0, 16)], ctlv.at[pl.ds(0, 16)])
  b1v = ctlv[pl.ds(0, 16)]
  _zero_ref(hist, 128)
  def h2(i, _):
    uk = load_skey(i) ^ _I32MIN
    m = lax.shift_right_logical(uk, 21) == b1v
    _hist_update(hist, lax.shift_right_logical(uk, 10) & _I32(0x7FF), m)
    return 0
  lax.fori_loop(0, nv, h2, 0)
  pltpu.sync_copy(hist.at[pl.ds(0, 2048)], HG.at[pl.ds(hg_off, 2048)])
  plsc.subcore_barrier()

  @pl.when(w == 0)
  def _():
    B2, cnt_gt = merge_and_find(2048, smem[0])
    P22 = lax.shift_left(smem[1], 11) | B2
    smem[0] = cnt_gt
    smem[1] = P22
    publish_ctl(1, P22)
  plsc.subcore_barrier()

  pltpu.sync_copy(CTL.at[pl.ds(16, 16)], ctlv.at[pl.ds(16, 16)])
  p22v = ctlv[pl.ds(16, 16)]
  _zero_ref(hist, 64)
  def h3(i, _):
    uk = load_skey(i) ^ _I32MIN
    m = lax.shift_right_logical(uk, 10) == p22v
    _hist_update(hist, uk & _I32(0x3FF), m)
    return 0
  lax.fori_loop(0, nv, h3, 0)
  pltpu.sync_copy(hist.at[pl.ds(0, 1024)], HG.at[pl.ds(hg_off, 1024)])
  plsc.subcore_barrier()

  @pl.when(w == 0)
  def _():
    B3, cnt_gt = merge_and_find(1024, smem[0])
    ukT = lax.shift_left(smem[1], 10) | B3
    sT = ukT ^ _I32MIN
    kt = K - cnt_gt
    publish_ctl(2, sT)
    publish_ctl(3, kt)
    vb = lax.shift_right_logical(B3, 4)
    lane = B3 & _I32(15)
    def tieloop(ww, toff):
      pltpu.sync_copy(HG.at[pl.ds(pl.multiple_of(ww * 2048, 8), 1024)],
                      row.at[pl.ds(0, 1024)])
      v = row[pl.ds(vb * 16, 16)]
      val = jnp.max(jnp.where(lax.iota(jnp.int32, 16) == lane, v, 0))
      ctlv[pl.ds(112, 16)] = jnp.full((16,), toff, jnp.int32)
      tof_off = pl.multiple_of(ww * 16, 8)
      pltpu.sync_copy(ctlv.at[pl.ds(112, 16)], TOF.at[pl.ds(tof_off, 16)])
      return toff + val
    lax.fori_loop(0, NW, tieloop, _I32(0))
  plsc.subcore_barrier()

  pltpu.sync_copy(CTL.at[pl.ds(32, 16)], ctlv.at[pl.ds(32, 16)])
  pltpu.sync_copy(CTL.at[pl.ds(48, 16)], ctlv.at[pl.ds(48, 16)])
  my_tof = pl.multiple_of(w * 16, 8)
  pltpu.sync_copy(TOF.at[pl.ds(my_tof, 16)], ctlv.at[pl.ds(112, 16)])
  sTv = ctlv[pl.ds(32, 16)]
  ktv = ctlv[pl.ds(48, 16)]
  tofv = ctlv[pl.ds(112, 16)]

  def sel_loop(i, carry):
    wp, ltc = carry
    o = i * 16
    skey = load_skey(i)
    gt = skey > sTv
    tie = skey == sTv
    tcum = plsc.cumsum(tie.astype(jnp.int32))
    keep = jnp.logical_and(tie, tofv + ltc + tcum - 1 < ktv)
    sel = jnp.logical_or(gt, keep)
    scum = plsc.cumsum(sel.astype(jnp.int32))
    pos = jnp.maximum(wp + scum - 1, 0)
    uk = skey ^ _I32MIN
    gidx = base + o + lax.iota(jnp.int32, 16)
    plsc.store_scatter(cak, [pos], uk, mask=sel)
    plsc.store_scatter(cai, [pos], gidx, mask=sel)
    ubits = jnp.where(skey >= 0, skey, skey ^ _I32(0x7FFFFFFF))
    keys[pl.ds(o, 16)] = jnp.where(sel, lax.bitcast_convert_type(ubits, jnp.float32),
                                   np.float32(0))
    return wp + jnp.max(scum), ltc + jnp.max(tcum)
  m, _ = lax.fori_loop(0, nv, sel_loop, (_I32(0), _I32(0)))

  @pl.when(jnp.logical_not(is15))
  def _():
    pltpu.sync_copy(keys.at[pl.ds(0, C)], out_hbm.at[pl.ds(base, C)])

  @pl.when(is15)
  def _():
    pltpu.sync_copy(keys, out_hbm.at[pl.ds(base, C15)])

  ctlv[pl.ds(96, 16)] = jnp.full((16,), m, jnp.int32)
  pltpu.sync_copy(ctlv.at[pl.ds(96, 16)], MC.at[pl.ds(my_tof, 16)])

  cs_base = pl.multiple_of(w * CAPS, 8)
  def pub(j, _):
    o = pl.multiple_of(j * 2504, 8)
    pltpu.sync_copy(cak.at[pl.ds(o, 2504)], CSK.at[pl.ds(cs_base + o, 2504)])
    pltpu.sync_copy(cai.at[pl.ds(o, 2504)], CSI.at[pl.ds(cs_base + o, 2504)])
    return 0
  lax.fori_loop(0, (m + 2503) // 2504, pub, 0)
  plsc.subcore_barrier()

  @pl.when(w == 0)
  def _():
    pltpu.sync_copy(MC, mbuf)

    def coll(ww, wp):
      mw = jnp.max(mbuf[pl.ds(pl.multiple_of(ww * 16, 8), 16)])
      cs_src = pl.multiple_of(ww * CAPS, 8)
      def chunk(j, wp2):
        o = pl.multiple_of(j * 1024, 8)
        pltpu.sync_copy(CSK.at[pl.ds(cs_src + o, 1024)], row.at[pl.ds(0, 1024)])
        pltpu.sync_copy(CSI.at[pl.ds(cs_src + o, 1024)],
                        row.at[pl.ds(1024, 1024)])
        rem = jnp.minimum(mw - j * 1024, 1024)
        def vc(i, _):
          pos = wp2 + i * 16 + lax.iota(jnp.int32, 16)
          plsc.store_scatter(cak, [pos], row[pl.ds(i * 16, 16)])
          plsc.store_scatter(cai, [pos], row[pl.ds(1024 + i * 16, 16)])
          return 0
        lax.fori_loop(0, (rem + 15) // 16, vc, 0)
        return wp2 + rem
      return lax.fori_loop(0, (mw + 1023) // 1024, chunk, wp)
    lax.fori_loop(1, NW, coll, jnp.max(mbuf[pl.ds(0, 16)]))

    bufs = [(cak, cai, cbk, cbi), (cbk, cbi, cak, cai), (cak, cai, cbk, cbi)]
    for p, (sh, bins) in enumerate([(0, 2048), (11, 2048), (22, 1024)]):
      src_k, src_i, dst_k, dst_i = bufs[p]
      _zero_ref(hist, bins // 16)
      def rh(i, _, sh=sh, bins=bins, src_k=src_k):
        d = lax.shift_right_logical(src_k[pl.ds(i * 16, 16)], sh) & _I32(bins - 1)
        _hist_update(hist, d, None)
        return 0
      lax.fori_loop(0, K // 16, rh, 0)

      def rb(i, carry):
        cs = plsc.cumsum(hist[pl.ds(i * 16, 16)])
        hist[pl.ds(i * 16, 16)] = K - (carry + cs)
        return carry + jnp.max(cs)
      lax.fori_loop(0, bins // 16, rb, _I32(0))

      def rs(i, _, sh=sh, bins=bins, src_k=src_k, src_i=src_i,
             dst_k=dst_k, dst_i=dst_i):
        kv = src_k[pl.ds(i * 16, 16)]
        iv = src_i[pl.ds(i * 16, 16)]
        d = lax.shift_right_logical(kv, sh) & _I32(bins - 1)
        b = plsc.load_gather(hist, [d])
        cnt, last = plsc.scan_count(d)
        pos = b + cnt - 1
        plsc.store_scatter(dst_k, [pos], kv)
        plsc.store_scatter(dst_i, [pos], iv)
        plsc.addupdate_scatter(hist, [d], cnt, mask=last)
        return 0
      lax.fori_loop(0, K // 16, rs, 0)

    pltpu.sync_copy(cbi.at[pl.ds(0, K)], win_hbm)


def kernel(act, active_units):
  mesh = plsc.VectorSubcoreMesh(core_axis_name="c", subcore_axis_name="s",
                                num_cores=1, num_subcores=NW)
  f = pl.kernel(
      _body,
      out_type=(
          jax.ShapeDtypeStruct((N,), jnp.float32),
          jax.ShapeDtypeStruct((K,), jnp.int32),
      ),
      mesh=mesh,
      compiler_params=pltpu.CompilerParams(needs_layout_passes=False),
      scratch_types=[
          pltpu.VMEM((C15,), jnp.float32),
          pltpu.VMEM((CH,), jnp.float32),
          pltpu.VMEM((2048,), jnp.int32),
          pltpu.VMEM((2048,), jnp.int32),
          pltpu.VMEM((128,), jnp.int32),
          pltpu.VMEM((256,), jnp.int32),
          pltpu.VMEM((CAP,), jnp.int32),
          pltpu.VMEM((CAP,), jnp.int32),
          pltpu.VMEM((CAP,), jnp.int32),
          pltpu.VMEM((CAP,), jnp.int32),
          pltpu.SMEM((8,), jnp.int32),
          pltpu.VMEM_SHARED((16 * 2048,), jnp.int32),
          pltpu.VMEM_SHARED((128,), jnp.int32),
          pltpu.VMEM_SHARED((256,), jnp.int32),
          pltpu.VMEM_SHARED((256,), jnp.int32),
          pltpu.VMEM_SHARED((16 * CAPS,), jnp.int32),
          pltpu.VMEM_SHARED((16 * CAPS,), jnp.int32),
      ],
  )
  return f(act, active_units)


# speedup vs baseline: 2.8470x; 1.0187x over previous
import jax
import jax.numpy as jnp
import numpy as np
from jax import lax
from jax.experimental import pallas as pl
from jax.experimental.pallas import tpu as pltpu
from jax.experimental.pallas import tpu_sc as plsc

N = 1_000_000
K = 10_000
NW = 16
C = 62_496
C15 = 62_560
NV = C // 16
NV15 = C15 // 16
CH = 672
CHV = CH // 16
CAP = 10_016
CAPS = 10_240

_I32MIN = np.int32(-(2**31))
_I32 = np.int32


def _zero_ref(ref, nv):
  z16 = lax.iota(jnp.int32, 16) * 0
  def z(i, _):
    ref[pl.ds(i * 16, 16)] = z16
    return 0
  lax.fori_loop(0, nv, z, 0, unroll=8)


def _hist_update(hist, bucket, mask):
  cnt, last = plsc.scan_count(bucket, mask)
  plsc.addupdate_scatter(hist, [bucket], cnt, mask=last)


def _body(act_hbm, active_hbm, out_hbm, win_hbm,
          keys, abuf, hist, row, ctlv, mbuf, cak, cai, cbk, cbi, smem,
          HG, CTL, TOF, MC, CSK, CSI):
  w = lax.axis_index("s")
  is15 = w == NW - 1
  base = pl.multiple_of(w * C, 8)
  hg_off = pl.multiple_of(w * 2048, 8)
  nv = jnp.where(is15, NV15, NV)

  @pl.when(jnp.logical_not(is15))
  def _():
    pltpu.sync_copy(act_hbm.at[pl.ds(base, C)], keys.at[pl.ds(0, C)])

  @pl.when(is15)
  def _():
    pltpu.sync_copy(act_hbm.at[pl.ds(base, C15)], keys)

  def keyify_vreg(o, a_act, a_active):
    a = a_act * a_active
    u = lax.bitcast_convert_type(a, jnp.int32)
    skey = jnp.where(u >= 0, u, u ^ _I32(0x7FFFFFFF))
    keys[pl.ds(o, 16)] = lax.bitcast_convert_type(skey, jnp.float32)

  def keyify_chunk(c, _):
    off = c * CH
    pltpu.sync_copy(active_hbm.at[pl.ds(base + off, CH)], abuf)
    def v(i, _):
      o = off + i * 16
      keyify_vreg(o, keys[pl.ds(o, 16)], abuf[pl.ds(i * 16, 16)])
      return 0
    lax.fori_loop(0, CHV, v, 0, unroll=4)
    return 0
  lax.fori_loop(0, C // CH, keyify_chunk, 0)

  @pl.when(is15)
  def _():
    pltpu.sync_copy(active_hbm.at[pl.ds(base + C, 64)], abuf.at[pl.ds(0, 64)])
    def v(i, _):
      o = C + i * 16
      keyify_vreg(o, keys[pl.ds(o, 16)], abuf[pl.ds(i * 16, 16)])
      return 0
    lax.fori_loop(0, 4, v, 0)

  def load_skey(i):
    return lax.bitcast_convert_type(keys[pl.ds(i * 16, 16)], jnp.int32)

  def merge_and_find(bins, cnt_gt):
    _zero_ref(hist, bins // 16)
    def addrow(ww, _):
      pltpu.sync_copy(HG.at[pl.ds(pl.multiple_of(ww * 2048, 8), bins)],
                      row.at[pl.ds(0, bins)])
      def av(i, _):
        hist[pl.ds(i * 16, 16)] = hist[pl.ds(i * 16, 16)] + row[pl.ds(i * 16, 16)]
        return 0
      lax.fori_loop(0, bins // 16, av, 0, unroll=8)
      return 0
    lax.fori_loop(0, NW, addrow, 0)

    def scan(j, carry):
      fb, fa, above = carry
      vb = bins // 16 - 1 - j
      v = hist[pl.ds(vb * 16, 16)]
      cs = plsc.cumsum(v)
      tot = jnp.max(cs)
      suf = above + tot - cs
      cond = jnp.logical_and(cnt_gt + suf < K, K <= cnt_gt + suf + v)
      lanei = lax.iota(jnp.int32, 16) + vb * 16
      fb = jnp.maximum(fb, jnp.max(jnp.where(cond, lanei, -1)))
      fa = jnp.maximum(fa, jnp.max(jnp.where(cond, suf, -1)))
      return fb, fa, above + tot
    B, above, _ = lax.fori_loop(0, bins // 16, scan,
                                (_I32(-1), _I32(-1), _I32(0)), unroll=4)
    return B, cnt_gt + above

  def publish_ctl(r, val):
    ctlv[pl.ds(r * 16, 16)] = jnp.full((16,), val, jnp.int32)
    pltpu.sync_copy(ctlv.at[pl.ds(r * 16, 16)], CTL.at[pl.ds(r * 16, 16)])

  _zero_ref(hist, 128)
  def h1(i, _):
    uk = load_skey(i) ^ _I32MIN
    _hist_update(hist, lax.shift_right_logical(uk, 21), jnp.full((16,), i < nv))
    return 0
  lax.fori_loop(0, NV15, h1, 0, unroll=4)
  pltpu.sync_copy(hist.at[pl.ds(0, 2048)], HG.at[pl.ds(hg_off, 2048)])
  plsc.subcore_barrier()

  @pl.when(w == 0)
  def _():
    B1, cnt_gt = merge_and_find(2048, _I32(0))
    smem[0] = cnt_gt
    smem[1] = B1
    publish_ctl(0, B1)
  plsc.subcore_barrier()

  pltpu.sync_copy(CTL.at[pl.ds(0, 16)], ctlv.at[pl.ds(0, 16)])
  b1v = ctlv[pl.ds(0, 16)]
  _zero_ref(hist, 128)
  def h2(i, _):
    uk = load_skey(i) ^ _I32MIN
    m = jnp.logical_and(lax.shift_right_logical(uk, 21) == b1v,
                        jnp.full((16,), i < nv))
    _hist_update(hist, lax.shift_right_logical(uk, 10) & _I32(0x7FF), m)
    return 0
  lax.fori_loop(0, NV15, h2, 0, unroll=4)
  pltpu.sync_copy(hist.at[pl.ds(0, 2048)], HG.at[pl.ds(hg_off, 2048)])
  plsc.subcore_barrier()

  @pl.when(w == 0)
  def _():
    B2, cnt_gt = merge_and_find(2048, smem[0])
    P22 = lax.shift_left(smem[1], 11) | B2
    smem[0] = cnt_gt
    smem[1] = P22
    publish_ctl(1, P22)
  plsc.subcore_barrier()

  pltpu.sync_copy(CTL.at[pl.ds(16, 16)], ctlv.at[pl.ds(16, 16)])
  p22v = ctlv[pl.ds(16, 16)]
  _zero_ref(hist, 64)
  def h3(i, _):
    uk = load_skey(i) ^ _I32MIN
    m = jnp.logical_and(lax.shift_right_logical(uk, 10) == p22v,
                        jnp.full((16,), i < nv))
    _hist_update(hist, uk & _I32(0x3FF), m)
    return 0
  lax.fori_loop(0, NV15, h3, 0, unroll=4)
  pltpu.sync_copy(hist.at[pl.ds(0, 1024)], HG.at[pl.ds(hg_off, 1024)])
  plsc.subcore_barrier()

  @pl.when(w == 0)
  def _():
    B3, cnt_gt = merge_and_find(1024, smem[0])
    ukT = lax.shift_left(smem[1], 10) | B3
    sT = ukT ^ _I32MIN
    kt = K - cnt_gt
    publish_ctl(2, sT)
    publish_ctl(3, kt)
    vb = lax.shift_right_logical(B3, 4)
    lane = B3 & _I32(15)
    def tieloop(ww, toff):
      pltpu.sync_copy(HG.at[pl.ds(pl.multiple_of(ww * 2048, 8), 1024)],
                      row.at[pl.ds(0, 1024)])
      v = row[pl.ds(vb * 16, 16)]
      val = jnp.max(jnp.where(lax.iota(jnp.int32, 16) == lane, v, 0))
      ctlv[pl.ds(112, 16)] = jnp.full((16,), toff, jnp.int32)
      tof_off = pl.multiple_of(ww * 16, 8)
      pltpu.sync_copy(ctlv.at[pl.ds(112, 16)], TOF.at[pl.ds(tof_off, 16)])
      return toff + val
    lax.fori_loop(0, NW, tieloop, _I32(0))
  plsc.subcore_barrier()

  pltpu.sync_copy(CTL.at[pl.ds(32, 16)], ctlv.at[pl.ds(32, 16)])
  pltpu.sync_copy(CTL.at[pl.ds(48, 16)], ctlv.at[pl.ds(48, 16)])
  my_tof = pl.multiple_of(w * 16, 8)
  pltpu.sync_copy(TOF.at[pl.ds(my_tof, 16)], ctlv.at[pl.ds(112, 16)])
  sTv = ctlv[pl.ds(32, 16)]
  ktv = ctlv[pl.ds(48, 16)]
  tofv = ctlv[pl.ds(112, 16)]

  def sel_loop(i, carry):
    wp, ltc = carry
    o = i * 16
    skey = load_skey(i)
    tailv = jnp.full((16,), i < nv)
    gt = jnp.logical_and(skey > sTv, tailv)
    tie = jnp.logical_and(skey == sTv, tailv)
    tcum = plsc.cumsum(tie.astype(jnp.int32))
    keep = jnp.logical_and(tie, tofv + ltc + tcum - 1 < ktv)
    sel = jnp.logical_or(gt, keep)
    scum = plsc.cumsum(sel.astype(jnp.int32))
    pos = jnp.maximum(wp + scum - 1, 0)
    uk = skey ^ _I32MIN
    gidx = base + o + lax.iota(jnp.int32, 16)
    plsc.store_scatter(cak, [pos], uk, mask=sel)
    plsc.store_scatter(cai, [pos], gidx, mask=sel)
    ubits = jnp.where(skey >= 0, skey, skey ^ _I32(0x7FFFFFFF))
    keys[pl.ds(o, 16)] = jnp.where(sel, lax.bitcast_convert_type(ubits, jnp.float32),
                                   np.float32(0))
    return wp + jnp.max(scum), ltc + jnp.max(tcum)
  m, _ = lax.fori_loop(0, NV15, sel_loop, (_I32(0), _I32(0)), unroll=2)

  @pl.when(jnp.logical_not(is15))
  def _():
    pltpu.sync_copy(keys.at[pl.ds(0, C)], out_hbm.at[pl.ds(base, C)])

  @pl.when(is15)
  def _():
    pltpu.sync_copy(keys, out_hbm.at[pl.ds(base, C15)])

  ctlv[pl.ds(96, 16)] = jnp.full((16,), m, jnp.int32)
  pltpu.sync_copy(ctlv.at[pl.ds(96, 16)], MC.at[pl.ds(my_tof, 16)])

  cs_base = pl.multiple_of(w * CAPS, 8)
  def pub(j, _):
    o = pl.multiple_of(j * 2504, 8)
    pltpu.sync_copy(cak.at[pl.ds(o, 2504)], CSK.at[pl.ds(cs_base + o, 2504)])
    pltpu.sync_copy(cai.at[pl.ds(o, 2504)], CSI.at[pl.ds(cs_base + o, 2504)])
    return 0
  lax.fori_loop(0, (m + 2503) // 2504, pub, 0)
  plsc.subcore_barrier()

  @pl.when(w == 0)
  def _():
    pltpu.sync_copy(MC, mbuf)

    def coll(ww, wp):
      mw = jnp.max(mbuf[pl.ds(pl.multiple_of(ww * 16, 8), 16)])
      cs_src = pl.multiple_of(ww * CAPS, 8)
      def chunk(j, wp2):
        o = pl.multiple_of(j * 1024, 8)
        pltpu.sync_copy(CSK.at[pl.ds(cs_src + o, 1024)], row.at[pl.ds(0, 1024)])
        pltpu.sync_copy(CSI.at[pl.ds(cs_src + o, 1024)],
                        row.at[pl.ds(1024, 1024)])
        rem = jnp.minimum(mw - j * 1024, 1024)
        def vc(i, _):
          pos = wp2 + i * 16 + lax.iota(jnp.int32, 16)
          plsc.store_scatter(cak, [pos], row[pl.ds(i * 16, 16)])
          plsc.store_scatter(cai, [pos], row[pl.ds(1024 + i * 16, 16)])
          return 0
        lax.fori_loop(0, (rem + 15) // 16, vc, 0)
        return wp2 + rem
      return lax.fori_loop(0, (mw + 1023) // 1024, chunk, wp)
    lax.fori_loop(1, NW, coll, jnp.max(mbuf[pl.ds(0, 16)]))

    bufs = [(cak, cai, cbk, cbi), (cbk, cbi, cak, cai), (cak, cai, cbk, cbi)]
    for p, (sh, bins) in enumerate([(0, 2048), (11, 2048), (22, 1024)]):
      src_k, src_i, dst_k, dst_i = bufs[p]
      _zero_ref(hist, bins // 16)
      def rh(i, _, sh=sh, bins=bins, src_k=src_k):
        d = lax.shift_right_logical(src_k[pl.ds(i * 16, 16)], sh) & _I32(bins - 1)
        _hist_update(hist, d, None)
        return 0
      lax.fori_loop(0, K // 16, rh, 0, unroll=4)

      def rb(i, carry):
        cs = plsc.cumsum(hist[pl.ds(i * 16, 16)])
        hist[pl.ds(i * 16, 16)] = K - (carry + cs)
        return carry + jnp.max(cs)
      lax.fori_loop(0, bins // 16, rb, _I32(0), unroll=4)

      def rs(i, _, sh=sh, bins=bins, src_k=src_k, src_i=src_i,
             dst_k=dst_k, dst_i=dst_i):
        kv = src_k[pl.ds(i * 16, 16)]
        iv = src_i[pl.ds(i * 16, 16)]
        d = lax.shift_right_logical(kv, sh) & _I32(bins - 1)
        b = plsc.load_gather(hist, [d])
        cnt, last = plsc.scan_count(d)
        pos = b + cnt - 1
        plsc.store_scatter(dst_k, [pos], kv)
        plsc.store_scatter(dst_i, [pos], iv)
        plsc.addupdate_scatter(hist, [d], cnt, mask=last)
        return 0
      lax.fori_loop(0, K // 16, rs, 0, unroll=4)

    pltpu.sync_copy(cbi.at[pl.ds(0, K)], win_hbm)


def kernel(act, active_units):
  mesh = plsc.VectorSubcoreMesh(core_axis_name="c", subcore_axis_name="s",
                                num_cores=1, num_subcores=NW)
  f = pl.kernel(
      _body,
      out_type=(
          jax.ShapeDtypeStruct((N,), jnp.float32),
          jax.ShapeDtypeStruct((K,), jnp.int32),
      ),
      mesh=mesh,
      compiler_params=pltpu.CompilerParams(needs_layout_passes=False),
      scratch_types=[
          pltpu.VMEM((C15,), jnp.float32),
          pltpu.VMEM((CH,), jnp.float32),
          pltpu.VMEM((2048,), jnp.int32),
          pltpu.VMEM((2048,), jnp.int32),
          pltpu.VMEM((128,), jnp.int32),
          pltpu.VMEM((256,), jnp.int32),
          pltpu.VMEM((CAP,), jnp.int32),
          pltpu.VMEM((CAP,), jnp.int32),
          pltpu.VMEM((CAP,), jnp.int32),
          pltpu.VMEM((CAP,), jnp.int32),
          pltpu.SMEM((8,), jnp.int32),
          pltpu.VMEM_SHARED((16 * 2048,), jnp.int32),
          pltpu.VMEM_SHARED((128,), jnp.int32),
          pltpu.VMEM_SHARED((256,), jnp.int32),
          pltpu.VMEM_SHARED((256,), jnp.int32),
          pltpu.VMEM_SHARED((16 * CAPS,), jnp.int32),
          pltpu.VMEM_SHARED((16 * CAPS,), jnp.int32),
      ],
  )
  return f(act, active_units)


# speedup vs baseline: 3.6930x; 1.2971x over previous
import jax
import jax.numpy as jnp
import numpy as np
from jax import lax
from jax.experimental import pallas as pl
from jax.experimental.pallas import tpu as pltpu
from jax.experimental.pallas import tpu_sc as plsc

N = 1_000_000
K = 10_000
NW = 16
C = 62_496
C15 = 62_560
NV = C // 16
NV15 = C15 // 16
CH = 672
CHV = CH // 16
CAP = 10_016
CAPS = 10_240

_I32MIN = np.int32(-(2**31))
_I32 = np.int32


def _zero_ref(ref, nv):
  z16 = lax.iota(jnp.int32, 16) * 0
  def z(i, _):
    ref[pl.ds(i * 16, 16)] = z16
    return 0
  lax.fori_loop(0, nv, z, 0, unroll=8)


def _hist_update(hist, bucket, mask):
  ones = lax.iota(jnp.int32, 16) * 0 + 1
  plsc.addupdate_scatter(hist, [bucket], ones, mask=mask)


def _body(act_hbm, active_hbm, out_hbm, win_hbm,
          keys, abuf, hist, row, ctlv, mbuf, cak, cai, cbk, cbi, smem,
          HG, CTL, TOF, MC, CSK, CSI):
  w = lax.axis_index("s")
  is15 = w == NW - 1
  base = pl.multiple_of(w * C, 8)
  hg_off = pl.multiple_of(w * 2048, 8)
  nv = jnp.where(is15, NV15, NV)

  @pl.when(jnp.logical_not(is15))
  def _():
    pltpu.sync_copy(act_hbm.at[pl.ds(base, C)], keys.at[pl.ds(0, C)])

  @pl.when(is15)
  def _():
    pltpu.sync_copy(act_hbm.at[pl.ds(base, C15)], keys)

  def keyify_vreg(o, a_act, a_active):
    a = a_act * a_active
    u = lax.bitcast_convert_type(a, jnp.int32)
    skey = jnp.where(u >= 0, u, u ^ _I32(0x7FFFFFFF))
    keys[pl.ds(o, 16)] = lax.bitcast_convert_type(skey, jnp.float32)

  def keyify_chunk(c, _):
    off = c * CH
    pltpu.sync_copy(active_hbm.at[pl.ds(base + off, CH)], abuf)
    def v(i, _):
      o = off + i * 16
      keyify_vreg(o, keys[pl.ds(o, 16)], abuf[pl.ds(i * 16, 16)])
      return 0
    lax.fori_loop(0, CHV, v, 0, unroll=4)
    return 0
  lax.fori_loop(0, C // CH, keyify_chunk, 0)

  @pl.when(is15)
  def _():
    pltpu.sync_copy(active_hbm.at[pl.ds(base + C, 64)], abuf.at[pl.ds(0, 64)])
    def v(i, _):
      o = C + i * 16
      keyify_vreg(o, keys[pl.ds(o, 16)], abuf[pl.ds(i * 16, 16)])
      return 0
    lax.fori_loop(0, 4, v, 0)

  def load_skey(i):
    return lax.bitcast_convert_type(keys[pl.ds(i * 16, 16)], jnp.int32)

  def merge_and_find(bins, cnt_gt):
    _zero_ref(hist, bins // 16)
    def addrow(ww, _):
      pltpu.sync_copy(HG.at[pl.ds(pl.multiple_of(ww * 2048, 8), bins)],
                      row.at[pl.ds(0, bins)])
      def av(i, _):
        hist[pl.ds(i * 16, 16)] = hist[pl.ds(i * 16, 16)] + row[pl.ds(i * 16, 16)]
        return 0
      lax.fori_loop(0, bins // 16, av, 0, unroll=8)
      return 0
    lax.fori_loop(0, NW, addrow, 0)

    def scan(j, carry):
      fb, fa, above = carry
      vb = bins // 16 - 1 - j
      v = hist[pl.ds(vb * 16, 16)]
      cs = plsc.cumsum(v)
      tot = jnp.max(cs)
      suf = above + tot - cs
      cond = jnp.logical_and(cnt_gt + suf < K, K <= cnt_gt + suf + v)
      lanei = lax.iota(jnp.int32, 16) + vb * 16
      fb = jnp.maximum(fb, jnp.max(jnp.where(cond, lanei, -1)))
      fa = jnp.maximum(fa, jnp.max(jnp.where(cond, suf, -1)))
      return fb, fa, above + tot
    B, above, _ = lax.fori_loop(0, bins // 16, scan,
                                (_I32(-1), _I32(-1), _I32(0)), unroll=4)
    return B, cnt_gt + above

  def publish_ctl(r, val):
    ctlv[pl.ds(r * 16, 16)] = jnp.full((16,), val, jnp.int32)
    pltpu.sync_copy(ctlv.at[pl.ds(r * 16, 16)], CTL.at[pl.ds(r * 16, 16)])

  _zero_ref(hist, 128)
  def h1(i, _):
    uk = load_skey(i) ^ _I32MIN
    _hist_update(hist, lax.shift_right_logical(uk, 21), jnp.full((16,), i < nv))
    return 0
  lax.fori_loop(0, NV15, h1, 0, unroll=4)
  pltpu.sync_copy(hist.at[pl.ds(0, 2048)], HG.at[pl.ds(hg_off, 2048)])
  plsc.subcore_barrier()

  @pl.when(w == 0)
  def _():
    B1, cnt_gt = merge_and_find(2048, _I32(0))
    smem[0] = cnt_gt
    smem[1] = B1
    publish_ctl(0, B1)
  plsc.subcore_barrier()

  pltpu.sync_copy(CTL.at[pl.ds(0, 16)], ctlv.at[pl.ds(0, 16)])
  b1v = ctlv[pl.ds(0, 16)]
  _zero_ref(hist, 128)
  def h2(i, _):
    uk = load_skey(i) ^ _I32MIN
    m = jnp.logical_and(lax.shift_right_logical(uk, 21) == b1v,
                        jnp.full((16,), i < nv))
    _hist_update(hist, lax.shift_right_logical(uk, 10) & _I32(0x7FF), m)
    return 0
  lax.fori_loop(0, NV15, h2, 0, unroll=4)
  pltpu.sync_copy(hist.at[pl.ds(0, 2048)], HG.at[pl.ds(hg_off, 2048)])
  plsc.subcore_barrier()

  @pl.when(w == 0)
  def _():
    B2, cnt_gt = merge_and_find(2048, smem[0])
    P22 = lax.shift_left(smem[1], 11) | B2
    smem[0] = cnt_gt
    smem[1] = P22
    publish_ctl(1, P22)
  plsc.subcore_barrier()

  pltpu.sync_copy(CTL.at[pl.ds(16, 16)], ctlv.at[pl.ds(16, 16)])
  p22v = ctlv[pl.ds(16, 16)]
  _zero_ref(hist, 64)
  def h3(i, _):
    uk = load_skey(i) ^ _I32MIN
    m = jnp.logical_and(lax.shift_right_logical(uk, 10) == p22v,
                        jnp.full((16,), i < nv))
    _hist_update(hist, uk & _I32(0x3FF), m)
    return 0
  lax.fori_loop(0, NV15, h3, 0, unroll=4)
  pltpu.sync_copy(hist.at[pl.ds(0, 1024)], HG.at[pl.ds(hg_off, 1024)])
  plsc.subcore_barrier()

  @pl.when(w == 0)
  def _():
    B3, cnt_gt = merge_and_find(1024, smem[0])
    ukT = lax.shift_left(smem[1], 10) | B3
    sT = ukT ^ _I32MIN
    kt = K - cnt_gt
    publish_ctl(2, sT)
    publish_ctl(3, kt)
    vb = lax.shift_right_logical(B3, 4)
    lane = B3 & _I32(15)
    def tieloop(ww, toff):
      pltpu.sync_copy(HG.at[pl.ds(pl.multiple_of(ww * 2048, 8), 1024)],
                      row.at[pl.ds(0, 1024)])
      v = row[pl.ds(vb * 16, 16)]
      val = jnp.max(jnp.where(lax.iota(jnp.int32, 16) == lane, v, 0))
      ctlv[pl.ds(112, 16)] = jnp.full((16,), toff, jnp.int32)
      tof_off = pl.multiple_of(ww * 16, 8)
      pltpu.sync_copy(ctlv.at[pl.ds(112, 16)], TOF.at[pl.ds(tof_off, 16)])
      return toff + val
    lax.fori_loop(0, NW, tieloop, _I32(0))
  plsc.subcore_barrier()

  pltpu.sync_copy(CTL.at[pl.ds(32, 16)], ctlv.at[pl.ds(32, 16)])
  pltpu.sync_copy(CTL.at[pl.ds(48, 16)], ctlv.at[pl.ds(48, 16)])
  my_tof = pl.multiple_of(w * 16, 8)
  pltpu.sync_copy(TOF.at[pl.ds(my_tof, 16)], ctlv.at[pl.ds(112, 16)])
  sTv = ctlv[pl.ds(32, 16)]
  ktv = ctlv[pl.ds(48, 16)]
  tofv = ctlv[pl.ds(112, 16)]

  def sel_loop(i, carry):
    wp, ltc = carry
    o = i * 16
    skey = load_skey(i)
    tailv = jnp.full((16,), i < nv)
    gt = jnp.logical_and(skey > sTv, tailv)
    tie = jnp.logical_and(skey == sTv, tailv)
    tcum = plsc.cumsum(tie.astype(jnp.int32))
    keep = jnp.logical_and(tie, tofv + ltc + tcum - 1 < ktv)
    sel = jnp.logical_or(gt, keep)
    scum = plsc.cumsum(sel.astype(jnp.int32))
    pos = jnp.maximum(wp + scum - 1, 0)
    uk = skey ^ _I32MIN
    gidx = base + o + lax.iota(jnp.int32, 16)
    plsc.store_scatter(cak, [pos], uk, mask=sel)
    plsc.store_scatter(cai, [pos], gidx, mask=sel)
    ubits = jnp.where(skey >= 0, skey, skey ^ _I32(0x7FFFFFFF))
    keys[pl.ds(o, 16)] = jnp.where(sel, lax.bitcast_convert_type(ubits, jnp.float32),
                                   np.float32(0))
    return wp + jnp.max(scum), ltc + jnp.max(tcum)
  m, _ = lax.fori_loop(0, NV15, sel_loop, (_I32(0), _I32(0)), unroll=2)

  @pl.when(jnp.logical_not(is15))
  def _():
    pltpu.sync_copy(keys.at[pl.ds(0, C)], out_hbm.at[pl.ds(base, C)])

  @pl.when(is15)
  def _():
    pltpu.sync_copy(keys, out_hbm.at[pl.ds(base, C15)])

  ctlv[pl.ds(96, 16)] = jnp.full((16,), m, jnp.int32)
  pltpu.sync_copy(ctlv.at[pl.ds(96, 16)], MC.at[pl.ds(my_tof, 16)])

  cs_base = pl.multiple_of(w * CAPS, 8)
  def pub(j, _):
    o = pl.multiple_of(j * 2504, 8)
    pltpu.sync_copy(cak.at[pl.ds(o, 2504)], CSK.at[pl.ds(cs_base + o, 2504)])
    pltpu.sync_copy(cai.at[pl.ds(o, 2504)], CSI.at[pl.ds(cs_base + o, 2504)])
    return 0
  lax.fori_loop(0, (m + 2503) // 2504, pub, 0)
  plsc.subcore_barrier()

  @pl.when(w == 0)
  def _():
    pltpu.sync_copy(MC, mbuf)

    def coll(ww, wp):
      mw = jnp.max(mbuf[pl.ds(pl.multiple_of(ww * 16, 8), 16)])
      cs_src = pl.multiple_of(ww * CAPS, 8)
      def chunk(j, wp2):
        o = pl.multiple_of(j * 1024, 8)
        pltpu.sync_copy(CSK.at[pl.ds(cs_src + o, 1024)], row.at[pl.ds(0, 1024)])
        pltpu.sync_copy(CSI.at[pl.ds(cs_src + o, 1024)],
                        row.at[pl.ds(1024, 1024)])
        rem = jnp.minimum(mw - j * 1024, 1024)
        def vc(i, _):
          pos = wp2 + i * 16 + lax.iota(jnp.int32, 16)
          plsc.store_scatter(cak, [pos], row[pl.ds(i * 16, 16)])
          plsc.store_scatter(cai, [pos], row[pl.ds(1024 + i * 16, 16)])
          return 0
        lax.fori_loop(0, (rem + 15) // 16, vc, 0)
        return wp2 + rem
      return lax.fori_loop(0, (mw + 1023) // 1024, chunk, wp)
    lax.fori_loop(1, NW, coll, jnp.max(mbuf[pl.ds(0, 16)]))

    bufs = [(cak, cai, cbk, cbi), (cbk, cbi, cak, cai), (cak, cai, cbk, cbi)]
    for p, (sh, bins) in enumerate([(0, 2048), (11, 2048), (22, 1024)]):
      src_k, src_i, dst_k, dst_i = bufs[p]
      _zero_ref(hist, bins // 16)
      def rh(i, _, sh=sh, bins=bins, src_k=src_k):
        d = lax.shift_right_logical(src_k[pl.ds(i * 16, 16)], sh) & _I32(bins - 1)
        _hist_update(hist, d, None)
        return 0
      lax.fori_loop(0, K // 16, rh, 0, unroll=4)

      def rb(i, carry):
        cs = plsc.cumsum(hist[pl.ds(i * 16, 16)])
        hist[pl.ds(i * 16, 16)] = K - (carry + cs)
        return carry + jnp.max(cs)
      lax.fori_loop(0, bins // 16, rb, _I32(0), unroll=4)

      def rs(i, _, sh=sh, bins=bins, src_k=src_k, src_i=src_i,
             dst_k=dst_k, dst_i=dst_i):
        kv = src_k[pl.ds(i * 16, 16)]
        iv = src_i[pl.ds(i * 16, 16)]
        d = lax.shift_right_logical(kv, sh) & _I32(bins - 1)
        b = plsc.load_gather(hist, [d])
        cnt, last = plsc.scan_count(d)
        pos = b + cnt - 1
        plsc.store_scatter(dst_k, [pos], kv)
        plsc.store_scatter(dst_i, [pos], iv)
        plsc.addupdate_scatter(hist, [d], cnt, mask=last)
        return 0
      lax.fori_loop(0, K // 16, rs, 0, unroll=4)

    pltpu.sync_copy(cbi.at[pl.ds(0, K)], win_hbm)


def kernel(act, active_units):
  mesh = plsc.VectorSubcoreMesh(core_axis_name="c", subcore_axis_name="s",
                                num_cores=1, num_subcores=NW)
  f = pl.kernel(
      _body,
      out_type=(
          jax.ShapeDtypeStruct((N,), jnp.float32),
          jax.ShapeDtypeStruct((K,), jnp.int32),
      ),
      mesh=mesh,
      compiler_params=pltpu.CompilerParams(needs_layout_passes=False),
      scratch_types=[
          pltpu.VMEM((C15,), jnp.float32),
          pltpu.VMEM((CH,), jnp.float32),
          pltpu.VMEM((2048,), jnp.int32),
          pltpu.VMEM((2048,), jnp.int32),
          pltpu.VMEM((128,), jnp.int32),
          pltpu.VMEM((256,), jnp.int32),
          pltpu.VMEM((CAP,), jnp.int32),
          pltpu.VMEM((CAP,), jnp.int32),
          pltpu.VMEM((CAP,), jnp.int32),
          pltpu.VMEM((CAP,), jnp.int32),
          pltpu.SMEM((8,), jnp.int32),
          pltpu.VMEM_SHARED((16 * 2048,), jnp.int32),
          pltpu.VMEM_SHARED((128,), jnp.int32),
          pltpu.VMEM_SHARED((256,), jnp.int32),
          pltpu.VMEM_SHARED((256,), jnp.int32),
          pltpu.VMEM_SHARED((16 * CAPS,), jnp.int32),
          pltpu.VMEM_SHARED((16 * CAPS,), jnp.int32),
      ],
  )
  return f(act, active_units)


# speedup vs baseline: 3.8144x; 1.0329x over previous
import jax
import jax.numpy as jnp
import numpy as np
from jax import lax
from jax.experimental import pallas as pl
from jax.experimental.pallas import tpu as pltpu
from jax.experimental.pallas import tpu_sc as plsc

N = 1_000_000
K = 10_000
NW = 16
C = 62_496
C15 = 62_560
NV = C // 16
NV15 = C15 // 16
CH = 672
CHV = CH // 16
CAP = 10_016
CAPS = 10_240

_I32MIN = np.int32(-(2**31))
_I32 = np.int32


def _zero_ref(ref, nv):
  z16 = lax.iota(jnp.int32, 16) * 0
  def z(i, _):
    ref[pl.ds(i * 16, 16)] = z16
    return 0
  lax.fori_loop(0, nv, z, 0, unroll=8)


def _hist_update(hist, bucket, mask):
  ones = lax.iota(jnp.int32, 16) * 0 + 1
  plsc.addupdate_scatter(hist, [bucket], ones, mask=mask)


def _body(act_hbm, active_hbm, out_hbm, win_hbm,
          keys, abuf, hist, row, ctlv, mbuf, cak, cai, cbk, cbi, smem,
          HG, CTL, TOF, MC, CSK, CSI):
  w = lax.axis_index("s")
  is15 = w == NW - 1
  base = pl.multiple_of(w * C, 8)
  hg_off = pl.multiple_of(w * 2048, 8)
  nv = jnp.where(is15, NV15, NV)

  @pl.when(jnp.logical_not(is15))
  def _():
    pltpu.sync_copy(act_hbm.at[pl.ds(base, C)], keys.at[pl.ds(0, C)])

  @pl.when(is15)
  def _():
    pltpu.sync_copy(act_hbm.at[pl.ds(base, C15)], keys)

  def keyify_vreg(o, a_act, a_active):
    a = a_act * a_active
    u = lax.bitcast_convert_type(a, jnp.int32)
    skey = jnp.where(u >= 0, u, u ^ _I32(0x7FFFFFFF))
    keys[pl.ds(o, 16)] = lax.bitcast_convert_type(skey, jnp.float32)
    uk = skey ^ _I32MIN
    _hist_update(hist, lax.shift_right_logical(uk, 21), None)

  _zero_ref(hist, 128)

  def keyify_chunk(c, _):
    off = c * CH
    pltpu.sync_copy(active_hbm.at[pl.ds(base + off, CH)], abuf)
    def v(i, _):
      o = off + i * 16
      keyify_vreg(o, keys[pl.ds(o, 16)], abuf[pl.ds(i * 16, 16)])
      return 0
    lax.fori_loop(0, CHV, v, 0, unroll=4)
    return 0
  lax.fori_loop(0, C // CH, keyify_chunk, 0)

  @pl.when(is15)
  def _():
    pltpu.sync_copy(active_hbm.at[pl.ds(base + C, 64)], abuf.at[pl.ds(0, 64)])
    def v(i, _):
      o = C + i * 16
      keyify_vreg(o, keys[pl.ds(o, 16)], abuf[pl.ds(i * 16, 16)])
      return 0
    lax.fori_loop(0, 4, v, 0)

  def load_skey(i):
    return lax.bitcast_convert_type(keys[pl.ds(i * 16, 16)], jnp.int32)

  def merge_and_find(bins, cnt_gt):
    _zero_ref(hist, bins // 16)
    def addrow(ww, _):
      pltpu.sync_copy(HG.at[pl.ds(pl.multiple_of(ww * 2048, 8), bins)],
                      row.at[pl.ds(0, bins)])
      def av(i, _):
        hist[pl.ds(i * 16, 16)] = hist[pl.ds(i * 16, 16)] + row[pl.ds(i * 16, 16)]
        return 0
      lax.fori_loop(0, bins // 16, av, 0, unroll=8)
      return 0
    lax.fori_loop(0, NW, addrow, 0)

    def scan(j, carry):
      fb, fa, above = carry
      vb = bins // 16 - 1 - j
      v = hist[pl.ds(vb * 16, 16)]
      cs = plsc.cumsum(v)
      tot = jnp.max(cs)
      suf = above + tot - cs
      cond = jnp.logical_and(cnt_gt + suf < K, K <= cnt_gt + suf + v)
      lanei = lax.iota(jnp.int32, 16) + vb * 16
      fb = jnp.maximum(fb, jnp.max(jnp.where(cond, lanei, -1)))
      fa = jnp.maximum(fa, jnp.max(jnp.where(cond, suf, -1)))
      return fb, fa, above + tot
    B, above, _ = lax.fori_loop(0, bins // 16, scan,
                                (_I32(-1), _I32(-1), _I32(0)), unroll=4)
    return B, cnt_gt + above

  def publish_ctl(r, val):
    ctlv[pl.ds(r * 16, 16)] = jnp.full((16,), val, jnp.int32)
    pltpu.sync_copy(ctlv.at[pl.ds(r * 16, 16)], CTL.at[pl.ds(r * 16, 16)])

  pltpu.sync_copy(hist.at[pl.ds(0, 2048)], HG.at[pl.ds(hg_off, 2048)])
  plsc.subcore_barrier()

  @pl.when(w == 0)
  def _():
    B1, cnt_gt = merge_and_find(2048, _I32(0))
    smem[0] = cnt_gt
    smem[1] = B1
    publish_ctl(0, B1)
  plsc.subcore_barrier()

  pltpu.sync_copy(CTL.at[pl.ds(0, 16)], ctlv.at[pl.ds(0, 16)])
  b1v = ctlv[pl.ds(0, 16)]
  _zero_ref(hist, 128)
  def h2(i, _):
    uk = load_skey(i) ^ _I32MIN
    m = jnp.logical_and(lax.shift_right_logical(uk, 21) == b1v,
                        jnp.full((16,), i < nv))
    _hist_update(hist, lax.shift_right_logical(uk, 10) & _I32(0x7FF), m)
    return 0
  lax.fori_loop(0, NV15, h2, 0, unroll=4)
  pltpu.sync_copy(hist.at[pl.ds(0, 2048)], HG.at[pl.ds(hg_off, 2048)])
  plsc.subcore_barrier()

  @pl.when(w == 0)
  def _():
    B2, cnt_gt = merge_and_find(2048, smem[0])
    P22 = lax.shift_left(smem[1], 11) | B2
    smem[0] = cnt_gt
    smem[1] = P22
    publish_ctl(1, P22)
  plsc.subcore_barrier()

  pltpu.sync_copy(CTL.at[pl.ds(16, 16)], ctlv.at[pl.ds(16, 16)])
  p22v = ctlv[pl.ds(16, 16)]
  _zero_ref(hist, 64)
  def h3(i, _):
    uk = load_skey(i) ^ _I32MIN
    m = jnp.logical_and(lax.shift_right_logical(uk, 10) == p22v,
                        jnp.full((16,), i < nv))
    _hist_update(hist, uk & _I32(0x3FF), m)
    return 0
  lax.fori_loop(0, NV15, h3, 0, unroll=4)
  pltpu.sync_copy(hist.at[pl.ds(0, 1024)], HG.at[pl.ds(hg_off, 1024)])
  plsc.subcore_barrier()

  @pl.when(w == 0)
  def _():
    B3, cnt_gt = merge_and_find(1024, smem[0])
    ukT = lax.shift_left(smem[1], 10) | B3
    sT = ukT ^ _I32MIN
    kt = K - cnt_gt
    publish_ctl(2, sT)
    publish_ctl(3, kt)
    vb = lax.shift_right_logical(B3, 4)
    lane = B3 & _I32(15)
    def tieloop(ww, toff):
      pltpu.sync_copy(HG.at[pl.ds(pl.multiple_of(ww * 2048, 8), 1024)],
                      row.at[pl.ds(0, 1024)])
      v = row[pl.ds(vb * 16, 16)]
      val = jnp.max(jnp.where(lax.iota(jnp.int32, 16) == lane, v, 0))
      ctlv[pl.ds(112, 16)] = jnp.full((16,), toff, jnp.int32)
      tof_off = pl.multiple_of(ww * 16, 8)
      pltpu.sync_copy(ctlv.at[pl.ds(112, 16)], TOF.at[pl.ds(tof_off, 16)])
      return toff + val
    lax.fori_loop(0, NW, tieloop, _I32(0))
  plsc.subcore_barrier()

  pltpu.sync_copy(CTL.at[pl.ds(32, 16)], ctlv.at[pl.ds(32, 16)])
  pltpu.sync_copy(CTL.at[pl.ds(48, 16)], ctlv.at[pl.ds(48, 16)])
  my_tof = pl.multiple_of(w * 16, 8)
  pltpu.sync_copy(TOF.at[pl.ds(my_tof, 16)], ctlv.at[pl.ds(112, 16)])
  sTv = ctlv[pl.ds(32, 16)]
  ktv = ctlv[pl.ds(48, 16)]
  tofv = ctlv[pl.ds(112, 16)]

  def sel_loop(i, carry):
    wp, ltc = carry
    o = i * 16
    skey = load_skey(i)
    tailv = jnp.full((16,), i < nv)
    gt = jnp.logical_and(skey > sTv, tailv)
    tie = jnp.logical_and(skey == sTv, tailv)
    tcum = plsc.cumsum(tie.astype(jnp.int32))
    keep = jnp.logical_and(tie, tofv + ltc + tcum - 1 < ktv)
    sel = jnp.logical_or(gt, keep)
    scum = plsc.cumsum(sel.astype(jnp.int32))
    pos = jnp.maximum(wp + scum - 1, 0)
    uk = skey ^ _I32MIN
    gidx = base + o + lax.iota(jnp.int32, 16)
    plsc.store_scatter(cak, [pos], uk, mask=sel)
    plsc.store_scatter(cai, [pos], gidx, mask=sel)
    ubits = jnp.where(skey >= 0, skey, skey ^ _I32(0x7FFFFFFF))
    keys[pl.ds(o, 16)] = jnp.where(sel, lax.bitcast_convert_type(ubits, jnp.float32),
                                   np.float32(0))
    return wp + jnp.max(scum), ltc + jnp.max(tcum)
  m, _ = lax.fori_loop(0, NV15, sel_loop, (_I32(0), _I32(0)), unroll=2)

  @pl.when(jnp.logical_not(is15))
  def _():
    pltpu.sync_copy(keys.at[pl.ds(0, C)], out_hbm.at[pl.ds(base, C)])

  @pl.when(is15)
  def _():
    pltpu.sync_copy(keys, out_hbm.at[pl.ds(base, C15)])

  ctlv[pl.ds(96, 16)] = jnp.full((16,), m, jnp.int32)
  pltpu.sync_copy(ctlv.at[pl.ds(96, 16)], MC.at[pl.ds(my_tof, 16)])

  cs_base = pl.multiple_of(w * CAPS, 8)
  def pub(j, _):
    o = pl.multiple_of(j * 2504, 8)
    pltpu.sync_copy(cak.at[pl.ds(o, 2504)], CSK.at[pl.ds(cs_base + o, 2504)])
    pltpu.sync_copy(cai.at[pl.ds(o, 2504)], CSI.at[pl.ds(cs_base + o, 2504)])
    return 0
  lax.fori_loop(0, (m + 2503) // 2504, pub, 0)
  plsc.subcore_barrier()

  @pl.when(w == 0)
  def _():
    pltpu.sync_copy(MC, mbuf)

    def coll(ww, wp):
      mw = jnp.max(mbuf[pl.ds(pl.multiple_of(ww * 16, 8), 16)])
      cs_src = pl.multiple_of(ww * CAPS, 8)
      def chunk(j, wp2):
        o = pl.multiple_of(j * 1024, 8)
        pltpu.sync_copy(CSK.at[pl.ds(cs_src + o, 1024)], row.at[pl.ds(0, 1024)])
        pltpu.sync_copy(CSI.at[pl.ds(cs_src + o, 1024)],
                        row.at[pl.ds(1024, 1024)])
        rem = jnp.minimum(mw - j * 1024, 1024)
        def vc(i, _):
          pos = wp2 + i * 16 + lax.iota(jnp.int32, 16)
          plsc.store_scatter(cak, [pos], row[pl.ds(i * 16, 16)])
          plsc.store_scatter(cai, [pos], row[pl.ds(1024 + i * 16, 16)])
          return 0
        lax.fori_loop(0, (rem + 15) // 16, vc, 0)
        return wp2 + rem
      return lax.fori_loop(0, (mw + 1023) // 1024, chunk, wp)
    lax.fori_loop(1, NW, coll, jnp.max(mbuf[pl.ds(0, 16)]))

    bufs = [(cak, cai, cbk, cbi), (cbk, cbi, cak, cai), (cak, cai, cbk, cbi)]
    for p, (sh, bins) in enumerate([(0, 2048), (11, 2048), (22, 1024)]):
      src_k, src_i, dst_k, dst_i = bufs[p]
      _zero_ref(hist, bins // 16)
      def rh(i, _, sh=sh, bins=bins, src_k=src_k):
        d = lax.shift_right_logical(src_k[pl.ds(i * 16, 16)], sh) & _I32(bins - 1)
        _hist_update(hist, d, None)
        return 0
      lax.fori_loop(0, K // 16, rh, 0, unroll=4)

      def rb(i, carry):
        cs = plsc.cumsum(hist[pl.ds(i * 16, 16)])
        hist[pl.ds(i * 16, 16)] = K - (carry + cs)
        return carry + jnp.max(cs)
      lax.fori_loop(0, bins // 16, rb, _I32(0), unroll=4)

      def rs(i, _, sh=sh, bins=bins, src_k=src_k, src_i=src_i,
             dst_k=dst_k, dst_i=dst_i):
        kv = src_k[pl.ds(i * 16, 16)]
        iv = src_i[pl.ds(i * 16, 16)]
        d = lax.shift_right_logical(kv, sh) & _I32(bins - 1)
        b = plsc.load_gather(hist, [d])
        cnt, last = plsc.scan_count(d)
        pos = b + cnt - 1
        plsc.store_scatter(dst_k, [pos], kv)
        plsc.store_scatter(dst_i, [pos], iv)
        plsc.addupdate_scatter(hist, [d], cnt, mask=last)
        return 0
      lax.fori_loop(0, K // 16, rs, 0, unroll=4)

    pltpu.sync_copy(cbi.at[pl.ds(0, K)], win_hbm)


def kernel(act, active_units):
  mesh = plsc.VectorSubcoreMesh(core_axis_name="c", subcore_axis_name="s",
                                num_cores=1, num_subcores=NW)
  f = pl.kernel(
      _body,
      out_type=(
          jax.ShapeDtypeStruct((N,), jnp.float32),
          jax.ShapeDtypeStruct((K,), jnp.int32),
      ),
      mesh=mesh,
      compiler_params=pltpu.CompilerParams(needs_layout_passes=False),
      scratch_types=[
          pltpu.VMEM((C15,), jnp.float32),
          pltpu.VMEM((CH,), jnp.float32),
          pltpu.VMEM((2048,), jnp.int32),
          pltpu.VMEM((2048,), jnp.int32),
          pltpu.VMEM((128,), jnp.int32),
          pltpu.VMEM((256,), jnp.int32),
          pltpu.VMEM((CAP,), jnp.int32),
          pltpu.VMEM((CAP,), jnp.int32),
          pltpu.VMEM((CAP,), jnp.int32),
          pltpu.VMEM((CAP,), jnp.int32),
          pltpu.SMEM((8,), jnp.int32),
          pltpu.VMEM_SHARED((16 * 2048,), jnp.int32),
          pltpu.VMEM_SHARED((128,), jnp.int32),
          pltpu.VMEM_SHARED((256,), jnp.int32),
          pltpu.VMEM_SHARED((256,), jnp.int32),
          pltpu.VMEM_SHARED((16 * CAPS,), jnp.int32),
          pltpu.VMEM_SHARED((16 * CAPS,), jnp.int32),
      ],
  )
  return f(act, active_units)


# speedup vs baseline: 3.8504x; 1.0094x over previous
import jax
import jax.numpy as jnp
import numpy as np
from jax import lax
from jax.experimental import pallas as pl
from jax.experimental.pallas import tpu as pltpu
from jax.experimental.pallas import tpu_sc as plsc

N = 1_000_000
K = 10_000
NW = 16
C = 62_496
C15 = 62_560
NV = C // 16
NV15 = C15 // 16
CH = 672
CHV = CH // 16
CAP = 10_016
CAPS = 10_240

_I32MIN = np.int32(-(2**31))
_I32 = np.int32


def _zero_ref(ref, nv):
  z16 = lax.iota(jnp.int32, 16) * 0
  def z(i, _):
    ref[pl.ds(i * 16, 16)] = z16
    return 0
  lax.fori_loop(0, nv, z, 0, unroll=8)


def _hist_update(hist, bucket, mask):
  ones = lax.iota(jnp.int32, 16) * 0 + 1
  plsc.addupdate_scatter(hist, [bucket], ones, mask=mask)


def _body(act_hbm, active_hbm, out_hbm, win_hbm,
          keys, abuf, hist, row, ctlv, mbuf, cak, cai, cbk, cbi, smem,
          HG, CTL, TOF, MC, CSK, CSI):
  w = lax.axis_index("s")
  is15 = w == NW - 1
  base = pl.multiple_of(w * C, 8)
  hg_off = pl.multiple_of(w * 2048, 8)
  nv = jnp.where(is15, NV15, NV)

  @pl.when(jnp.logical_not(is15))
  def _():
    pltpu.sync_copy(act_hbm.at[pl.ds(base, C)], keys.at[pl.ds(0, C)])

  @pl.when(is15)
  def _():
    pltpu.sync_copy(act_hbm.at[pl.ds(base, C15)], keys)

  def keyify_vreg(o, a_act, a_active):
    a = a_act * a_active
    u = lax.bitcast_convert_type(a, jnp.int32)
    skey = jnp.where(u >= 0, u, u ^ _I32(0x7FFFFFFF))
    keys[pl.ds(o, 16)] = lax.bitcast_convert_type(skey, jnp.float32)
    uk = skey ^ _I32MIN
    _hist_update(hist, lax.shift_right_logical(uk, 21), None)

  _zero_ref(hist, 128)

  def keyify_chunk(c, _):
    off = c * CH
    pltpu.sync_copy(active_hbm.at[pl.ds(base + off, CH)], abuf)
    def v(i, _):
      o = off + i * 16
      keyify_vreg(o, keys[pl.ds(o, 16)], abuf[pl.ds(i * 16, 16)])
      return 0
    lax.fori_loop(0, CHV, v, 0, unroll=4)
    return 0
  lax.fori_loop(0, C // CH, keyify_chunk, 0)

  @pl.when(is15)
  def _():
    pltpu.sync_copy(active_hbm.at[pl.ds(base + C, 64)], abuf.at[pl.ds(0, 64)])
    def v(i, _):
      o = C + i * 16
      keyify_vreg(o, keys[pl.ds(o, 16)], abuf[pl.ds(i * 16, 16)])
      return 0
    lax.fori_loop(0, 4, v, 0)

  def load_skey(i):
    return lax.bitcast_convert_type(keys[pl.ds(i * 16, 16)], jnp.int32)

  def merge_and_find(bins, cnt_gt):
    _zero_ref(hist, bins // 16)
    def addrow(ww, _):
      pltpu.sync_copy(HG.at[pl.ds(pl.multiple_of(ww * 2048, 8), bins)],
                      row.at[pl.ds(0, bins)])
      def av(i, _):
        hist[pl.ds(i * 16, 16)] = hist[pl.ds(i * 16, 16)] + row[pl.ds(i * 16, 16)]
        return 0
      lax.fori_loop(0, bins // 16, av, 0, unroll=8)
      return 0
    lax.fori_loop(0, NW, addrow, 0)

    def scan(j, carry):
      fb, fa, above = carry
      vb = bins // 16 - 1 - j
      v = hist[pl.ds(vb * 16, 16)]
      cs = plsc.cumsum(v)
      tot = cs[15]
      suf = above + tot - cs
      cond = jnp.logical_and(cnt_gt + suf < K, K <= cnt_gt + suf + v)
      lanei = lax.iota(jnp.int32, 16) + vb * 16
      fb = jnp.maximum(fb, jnp.max(jnp.where(cond, lanei, -1)))
      fa = jnp.maximum(fa, jnp.max(jnp.where(cond, suf, -1)))
      return fb, fa, above + tot
    B, above, _ = lax.fori_loop(0, bins // 16, scan,
                                (_I32(-1), _I32(-1), _I32(0)), unroll=4)
    return B, cnt_gt + above

  def publish_ctl(r, val):
    ctlv[pl.ds(r * 16, 16)] = jnp.full((16,), val, jnp.int32)
    pltpu.sync_copy(ctlv.at[pl.ds(r * 16, 16)], CTL.at[pl.ds(r * 16, 16)])

  pltpu.sync_copy(hist.at[pl.ds(0, 2048)], HG.at[pl.ds(hg_off, 2048)])
  plsc.subcore_barrier()

  @pl.when(w == 0)
  def _():
    B1, cnt_gt = merge_and_find(2048, _I32(0))
    smem[0] = cnt_gt
    smem[1] = B1
    publish_ctl(0, B1)
  plsc.subcore_barrier()

  pltpu.sync_copy(CTL.at[pl.ds(0, 16)], ctlv.at[pl.ds(0, 16)])
  b1v = ctlv[pl.ds(0, 16)]
  _zero_ref(hist, 128)
  def h2(i, _):
    uk = load_skey(i) ^ _I32MIN
    m = jnp.logical_and(lax.shift_right_logical(uk, 21) == b1v,
                        jnp.full((16,), i < nv))
    _hist_update(hist, lax.shift_right_logical(uk, 10) & _I32(0x7FF), m)
    return 0
  lax.fori_loop(0, NV15, h2, 0, unroll=4)
  pltpu.sync_copy(hist.at[pl.ds(0, 2048)], HG.at[pl.ds(hg_off, 2048)])
  plsc.subcore_barrier()

  @pl.when(w == 0)
  def _():
    B2, cnt_gt = merge_and_find(2048, smem[0])
    P22 = lax.shift_left(smem[1], 11) | B2
    smem[0] = cnt_gt
    smem[1] = P22
    publish_ctl(1, P22)
  plsc.subcore_barrier()

  pltpu.sync_copy(CTL.at[pl.ds(16, 16)], ctlv.at[pl.ds(16, 16)])
  p22v = ctlv[pl.ds(16, 16)]
  _zero_ref(hist, 64)
  def h3(i, _):
    uk = load_skey(i) ^ _I32MIN
    m = jnp.logical_and(lax.shift_right_logical(uk, 10) == p22v,
                        jnp.full((16,), i < nv))
    _hist_update(hist, uk & _I32(0x3FF), m)
    return 0
  lax.fori_loop(0, NV15, h3, 0, unroll=4)
  pltpu.sync_copy(hist.at[pl.ds(0, 1024)], HG.at[pl.ds(hg_off, 1024)])
  plsc.subcore_barrier()

  @pl.when(w == 0)
  def _():
    B3, cnt_gt = merge_and_find(1024, smem[0])
    ukT = lax.shift_left(smem[1], 10) | B3
    sT = ukT ^ _I32MIN
    kt = K - cnt_gt
    publish_ctl(2, sT)
    publish_ctl(3, kt)
    vb = lax.shift_right_logical(B3, 4)
    lane = B3 & _I32(15)
    def tieloop(ww, toff):
      pltpu.sync_copy(HG.at[pl.ds(pl.multiple_of(ww * 2048, 8), 1024)],
                      row.at[pl.ds(0, 1024)])
      v = row[pl.ds(vb * 16, 16)]
      val = jnp.max(jnp.where(lax.iota(jnp.int32, 16) == lane, v, 0))
      ctlv[pl.ds(112, 16)] = jnp.full((16,), toff, jnp.int32)
      tof_off = pl.multiple_of(ww * 16, 8)
      pltpu.sync_copy(ctlv.at[pl.ds(112, 16)], TOF.at[pl.ds(tof_off, 16)])
      return toff + val
    lax.fori_loop(0, NW, tieloop, _I32(0))
  plsc.subcore_barrier()

  pltpu.sync_copy(CTL.at[pl.ds(32, 16)], ctlv.at[pl.ds(32, 16)])
  pltpu.sync_copy(CTL.at[pl.ds(48, 16)], ctlv.at[pl.ds(48, 16)])
  my_tof = pl.multiple_of(w * 16, 8)
  pltpu.sync_copy(TOF.at[pl.ds(my_tof, 16)], ctlv.at[pl.ds(112, 16)])
  sTv = ctlv[pl.ds(32, 16)]
  ktv = ctlv[pl.ds(48, 16)]
  tofv = ctlv[pl.ds(112, 16)]

  def sel_loop(i, carry):
    wp, ltc = carry
    o = i * 16
    skey = load_skey(i)
    tailv = jnp.full((16,), i < nv)
    gt = jnp.logical_and(skey > sTv, tailv)
    tie = jnp.logical_and(skey == sTv, tailv)
    tcum = plsc.cumsum(tie.astype(jnp.int32))
    keep = jnp.logical_and(tie, tofv + ltc + tcum - 1 < ktv)
    sel = jnp.logical_or(gt, keep)
    scum = plsc.cumsum(sel.astype(jnp.int32))
    pos = jnp.maximum(wp + scum - 1, 0)
    uk = skey ^ _I32MIN
    gidx = base + o + lax.iota(jnp.int32, 16)
    plsc.store_scatter(cak, [pos], uk, mask=sel)
    plsc.store_scatter(cai, [pos], gidx, mask=sel)
    ubits = jnp.where(skey >= 0, skey, skey ^ _I32(0x7FFFFFFF))
    keys[pl.ds(o, 16)] = jnp.where(sel, lax.bitcast_convert_type(ubits, jnp.float32),
                                   np.float32(0))
    return wp + scum[15], ltc + tcum[15]
  m, _ = lax.fori_loop(0, NV15, sel_loop, (_I32(0), _I32(0)), unroll=2)

  @pl.when(jnp.logical_not(is15))
  def _():
    pltpu.sync_copy(keys.at[pl.ds(0, C)], out_hbm.at[pl.ds(base, C)])

  @pl.when(is15)
  def _():
    pltpu.sync_copy(keys, out_hbm.at[pl.ds(base, C15)])

  ctlv[pl.ds(96, 16)] = jnp.full((16,), m, jnp.int32)
  pltpu.sync_copy(ctlv.at[pl.ds(96, 16)], MC.at[pl.ds(my_tof, 16)])

  cs_base = pl.multiple_of(w * CAPS, 8)
  def pub(j, _):
    o = pl.multiple_of(j * 2504, 8)
    pltpu.sync_copy(cak.at[pl.ds(o, 2504)], CSK.at[pl.ds(cs_base + o, 2504)])
    pltpu.sync_copy(cai.at[pl.ds(o, 2504)], CSI.at[pl.ds(cs_base + o, 2504)])
    return 0
  lax.fori_loop(0, (m + 2503) // 2504, pub, 0)
  plsc.subcore_barrier()

  @pl.when(w == 0)
  def _():
    pltpu.sync_copy(MC, mbuf)

    def coll(ww, wp):
      mw = jnp.max(mbuf[pl.ds(pl.multiple_of(ww * 16, 8), 16)])
      cs_src = pl.multiple_of(ww * CAPS, 8)
      def chunk(j, wp2):
        o = pl.multiple_of(j * 1024, 8)
        pltpu.sync_copy(CSK.at[pl.ds(cs_src + o, 1024)], row.at[pl.ds(0, 1024)])
        pltpu.sync_copy(CSI.at[pl.ds(cs_src + o, 1024)],
                        row.at[pl.ds(1024, 1024)])
        rem = jnp.minimum(mw - j * 1024, 1024)
        def vc(i, _):
          pos = wp2 + i * 16 + lax.iota(jnp.int32, 16)
          plsc.store_scatter(cak, [pos], row[pl.ds(i * 16, 16)])
          plsc.store_scatter(cai, [pos], row[pl.ds(1024 + i * 16, 16)])
          return 0
        lax.fori_loop(0, (rem + 15) // 16, vc, 0)
        return wp2 + rem
      return lax.fori_loop(0, (mw + 1023) // 1024, chunk, wp)
    lax.fori_loop(1, NW, coll, jnp.max(mbuf[pl.ds(0, 16)]))

    bufs = [(cak, cai, cbk, cbi), (cbk, cbi, cak, cai), (cak, cai, cbk, cbi)]
    for p, (sh, bins) in enumerate([(0, 2048), (11, 2048), (22, 1024)]):
      src_k, src_i, dst_k, dst_i = bufs[p]
      _zero_ref(hist, bins // 16)
      def rh(i, _, sh=sh, bins=bins, src_k=src_k):
        d = lax.shift_right_logical(src_k[pl.ds(i * 16, 16)], sh) & _I32(bins - 1)
        _hist_update(hist, d, None)
        return 0
      lax.fori_loop(0, K // 16, rh, 0, unroll=4)

      def rb(i, carry):
        cs = plsc.cumsum(hist[pl.ds(i * 16, 16)])
        hist[pl.ds(i * 16, 16)] = K - (carry + cs)
        return carry + cs[15]
      lax.fori_loop(0, bins // 16, rb, _I32(0), unroll=4)

      def rs(i, _, sh=sh, bins=bins, src_k=src_k, src_i=src_i,
             dst_k=dst_k, dst_i=dst_i):
        kv = src_k[pl.ds(i * 16, 16)]
        iv = src_i[pl.ds(i * 16, 16)]
        d = lax.shift_right_logical(kv, sh) & _I32(bins - 1)
        b = plsc.load_gather(hist, [d])
        cnt, last = plsc.scan_count(d)
        pos = b + cnt - 1
        plsc.store_scatter(dst_k, [pos], kv)
        plsc.store_scatter(dst_i, [pos], iv)
        plsc.addupdate_scatter(hist, [d], cnt, mask=last)
        return 0
      lax.fori_loop(0, K // 16, rs, 0, unroll=4)

    pltpu.sync_copy(cbi.at[pl.ds(0, K)], win_hbm)


def kernel(act, active_units):
  mesh = plsc.VectorSubcoreMesh(core_axis_name="c", subcore_axis_name="s",
                                num_cores=1, num_subcores=NW)
  f = pl.kernel(
      _body,
      out_type=(
          jax.ShapeDtypeStruct((N,), jnp.float32),
          jax.ShapeDtypeStruct((K,), jnp.int32),
      ),
      mesh=mesh,
      compiler_params=pltpu.CompilerParams(needs_layout_passes=False),
      scratch_types=[
          pltpu.VMEM((C15,), jnp.float32),
          pltpu.VMEM((CH,), jnp.float32),
          pltpu.VMEM((2048,), jnp.int32),
          pltpu.VMEM((2048,), jnp.int32),
          pltpu.VMEM((128,), jnp.int32),
          pltpu.VMEM((256,), jnp.int32),
          pltpu.VMEM((CAP,), jnp.int32),
          pltpu.VMEM((CAP,), jnp.int32),
          pltpu.VMEM((CAP,), jnp.int32),
          pltpu.VMEM((CAP,), jnp.int32),
          pltpu.SMEM((8,), jnp.int32),
          pltpu.VMEM_SHARED((16 * 2048,), jnp.int32),
          pltpu.VMEM_SHARED((128,), jnp.int32),
          pltpu.VMEM_SHARED((256,), jnp.int32),
          pltpu.VMEM_SHARED((256,), jnp.int32),
          pltpu.VMEM_SHARED((16 * CAPS,), jnp.int32),
          pltpu.VMEM_SHARED((16 * CAPS,), jnp.int32),
      ],
  )
  return f(act, active_units)
